# baseline (speedup 1.0000x reference)
import functools

import jax
import jax.numpy as jnp
from jax import lax
from jax.experimental import pallas as pl
from jax.experimental.pallas import tpu as pltpu
from jax.experimental.pallas import tpu_sc as plsc

N = 10000
M = 128
MY = 16
E = 320000
MAX_ITER = 30
GAMMA = 0.8
EPS_F = 1e-6

N_PAD = 10240
CH = 128
N_TILES = 32
CPT = 80
E_PAD = N_TILES * CPT * CH
E_ALLOC = E_PAD + 2 * CH
RPT = N_PAD // 16

_MESH = plsc.VectorSubcoreMesh(core_axis_name="c", subcore_axis_name="s")


@functools.partial(
    pl.kernel,
    out_type=jax.ShapeDtypeStruct((4 * N_PAD, M), jnp.float32),
    mesh=_MESH,
    scratch_types=[
        pltpu.VMEM_SHARED((N_PAD, M), jnp.float32),
        pltpu.VMEM((CH,), jnp.int32),
        pltpu.VMEM((CH, M), jnp.float32),
        pltpu.VMEM((CH, M), jnp.float32),
    ],
)
def _deg_kernel(srcp, dstp, deg_out, acc, idx_v, ones_v, bnc_v):
    cid = lax.axis_index("c")
    sid = lax.axis_index("s")
    wid = sid * 2 + cid
    onerow = jnp.ones((16,), jnp.float32)
    zrow = jnp.zeros((16,), jnp.float32)

    def fill_ones(r, carry):
        for k in range(M // 16):
            ones_v[r, pl.ds(k * 16, 16)] = onerow
        return carry

    lax.fori_loop(0, CH, fill_ones, 0)

    def fill_bnc_zero(r, carry):
        for k in range(M // 16):
            bnc_v[r, pl.ds(k * 16, 16)] = zrow
        return carry

    base_r = sid * RPT

    def zero_acc(b, carry):
        pltpu.sync_copy(bnc_v, acc.at[pl.ds(base_r + b * CH, CH)])
        return carry

    def count_phase(idx_hbm, out_base):
        lax.fori_loop(0, CH, fill_bnc_zero, 0)
        lax.fori_loop(0, RPT // CH, zero_acc, 0)
        plsc.subcore_barrier()

        def edge_step(j, carry):
            base = (wid * CPT + j) * CH
            pltpu.sync_copy(idx_hbm.at[pl.ds(base, CH)], idx_v)
            pltpu.sync_copy(ones_v, acc.at[idx_v], add=True)
            return carry

        lax.fori_loop(0, CPT, edge_step, 0)
        plsc.subcore_barrier()

        def writeback(b, carry):
            off = base_r + b * CH
            pltpu.sync_copy(acc.at[pl.ds(off, CH)], bnc_v)
            pltpu.sync_copy(bnc_v, deg_out.at[pl.ds(out_base + off, CH)])
            return carry

        lax.fori_loop(0, RPT // CH, writeback, 0)
        plsc.subcore_barrier()

    count_phase(srcp, cid * N_PAD)
    count_phase(dstp, 2 * N_PAD + cid * N_PAD)


FPT = M // N_TILES
SG_E = 8192
NSG = E_PAD // SG_E


@functools.partial(
    pl.kernel,
    out_type=jax.ShapeDtypeStruct((M * N_PAD,), jnp.float32),
    mesh=_MESH,
    scratch_types=[
        pltpu.VMEM((FPT * N_PAD,), jnp.float32),
        pltpu.VMEM((FPT * N_PAD,), jnp.float32),
        pltpu.VMEM((SG_E,), jnp.int32),
        pltpu.VMEM((SG_E,), jnp.int32),
        pltpu.VMEM((SG_E,), jnp.int32),
        pltpu.VMEM((SG_E,), jnp.int32),
        pltpu.SemaphoreType.DMA,
        pltpu.SemaphoreType.DMA,
    ],
    compiler_params=pltpu.CompilerParams(needs_layout_passes=False),
)
def _spmm_kernel(zflat, srcp, dstp, pflat, zloc, ploc, ib0s, ib0d,
                 ib1s, ib1d, si0, si1):
    cid = lax.axis_index("c")
    sid = lax.axis_index("s")
    wid = sid * 2 + cid
    IBS, IBD, SI = [ib0s, ib1s], [ib0d, ib1d], [si0, si1]
    zrow = jnp.zeros((16,), jnp.float32)
    my_off = wid * FPT * N_PAD

    pltpu.sync_copy(zflat.at[pl.ds(my_off, FPT * N_PAD)], zloc)

    def fill_zero(r, carry):
        ploc[pl.ds(r * 16, 16)] = zrow
        return carry

    lax.fori_loop(0, FPT * N_PAD // 16, fill_zero, 0)

    def issue(s, sg):
        base = sg * SG_E
        return (pltpu.async_copy(srcp.at[pl.ds(base, SG_E)], IBS[s], SI[s]),
                pltpu.async_copy(dstp.at[pl.ds(base, SG_E)], IBD[s], SI[s]))

    offs = [jnp.full((16,), j * N_PAD, jnp.int32) for j in range(FPT)]
    d = [issue(0, 0), issue(1, 1)]
    for sg in range(NSG):
        s = sg % 2
        d[s][0].wait()
        d[s][1].wait()
        ibs, ibd = IBS[s], IBD[s]

        def edge_group(g, carry):
            off = pl.multiple_of(g * 128, 16)
            svs = [ibs[pl.ds(off + 16 * u, 16)] for u in range(8)]
            dvs = [ibd[pl.ds(off + 16 * u, 16)] for u in range(8)]
            gidx = [svs[u] + offs[j] for u in range(8) for j in range(FPT)]
            vals = [plsc.load_gather(zloc, [gi]) for gi in gidx]
            for u in range(8):
                for j in range(FPT):
                    plsc.addupdate_scatter(ploc, [dvs[u] + offs[j]],
                                           vals[u * FPT + j])
            return carry

        lax.fori_loop(0, SG_E // 128, edge_group, 0)
        if sg + 2 < NSG:
            d[s] = issue(s, sg + 2)

    pltpu.sync_copy(ploc, pflat.at[pl.ds(my_off, FPT * N_PAD)])


def _prep_body(xb, fw, do0, do1, di0, di1, xa_o, cb_o, gf_o):
    dego_t = (do0[...] + do1[...]).T
    degi_t = (di0[...] + di1[...]).T
    a = lax.rsqrt(jnp.maximum(dego_t[0:1, :], 1.0))
    b = lax.rsqrt(jnp.maximum(degi_t[0:1, :], 1.0))
    c = GAMMA * a * b
    xa_o[...] = xb[...] * a
    cb_o[...] = jnp.broadcast_to(c, (M, M))
    g = lax.dot_general(fw[...], fw[...], (((0,), (0,)), ((), ())),
                        precision=lax.Precision.HIGHEST)
    nrm = jnp.sqrt(jnp.sum(g * g))
    gf_o[...] = g / (nrm + EPS_F)


def _prep_call(xp, f_w, degs):
    nb = N_PAD // M
    return pl.pallas_call(
        _prep_body,
        grid=(nb,),
        in_specs=[
            pl.BlockSpec((M, M), lambda i: (0, i)),
            pl.BlockSpec((M, M), lambda i: (0, 0)),
            pl.BlockSpec((M, M), lambda i: (i, 0)),
            pl.BlockSpec((M, M), lambda i, _nb=nb: (i + _nb, 0)),
            pl.BlockSpec((M, M), lambda i, _nb=nb: (i + 2 * _nb, 0)),
            pl.BlockSpec((M, M), lambda i, _nb=nb: (i + 3 * _nb, 0)),
        ],
        out_specs=[
            pl.BlockSpec((M, M), lambda i: (0, i)),
            pl.BlockSpec((M, M), lambda i: (0, i)),
            pl.BlockSpec((M, M), lambda i: (0, 0)),
        ],
        out_shape=[
            jax.ShapeDtypeStruct((M, N_PAD), jnp.float32),
            jax.ShapeDtypeStruct((M, N_PAD), jnp.float32),
            jax.ShapeDtypeStruct((M, M), jnp.float32),
        ],
        compiler_params=pltpu.CompilerParams(
            dimension_semantics=("arbitrary",)),
    )(xp, f_w, degs, degs, degs, degs)


def _update_body(p, cbk, xak, gf, z_o):
    acc = p[...] * cbk[...]
    z_o[...] = lax.dot_general(
        gf[...], acc, (((1,), (0,)), ((), ())),
        precision=lax.Precision.HIGHEST) + xak[...]


def _update_call(pf, cb, xa, gf):
    rb = 2048
    nb = N_PAD // rb
    return pl.pallas_call(
        _update_body,
        grid=(nb,),
        in_specs=[
            pl.BlockSpec((M, rb), lambda i: (0, i)),
            pl.BlockSpec((M, rb), lambda i: (0, i)),
            pl.BlockSpec((M, rb), lambda i: (0, i)),
            pl.BlockSpec((M, M), lambda i: (0, 0)),
        ],
        out_specs=pl.BlockSpec((M, rb), lambda i: (0, i)),
        out_shape=jax.ShapeDtypeStruct((M, N_PAD), jnp.float32),
        compiler_params=pltpu.CompilerParams(
            dimension_semantics=("arbitrary",)),
    )(pf, cb, xa, gf)


def _final_body(zb, bw, o_ref):
    z = zb[...]
    nrm = jnp.maximum(jnp.sqrt(jnp.sum(z * z, axis=0, keepdims=True)), 1e-12)
    zn = z / nrm
    o_ref[...] = lax.dot_general(zn, bw[...], (((0,), (1,)), ((), ())),
                                 precision=lax.Precision.HIGHEST)


def _final_call(z, b_w):
    rb = 1024
    nb = N_PAD // rb
    return pl.pallas_call(
        _final_body,
        grid=(nb,),
        in_specs=[
            pl.BlockSpec((M, rb), lambda i: (0, i)),
            pl.BlockSpec((MY, M), lambda i: (0, 0)),
        ],
        out_specs=pl.BlockSpec((rb, MY), lambda i: (i, 0)),
        out_shape=jax.ShapeDtypeStruct((N, MY), jnp.float32),
        compiler_params=pltpu.CompilerParams(
            dimension_semantics=("arbitrary",)),
    )(z, b_w)


def kernel(X, edge_index, F_w, B_w):
    src = edge_index[0]
    dst = edge_index[1]
    pad = jnp.full((E_PAD - E,), N, dtype=jnp.int32)
    srcp = jnp.concatenate([src, pad])
    dstp = jnp.concatenate([dst, pad])
    xp = jnp.pad(X, ((0, 0), (0, N_PAD - N)))

    degs = _deg_kernel(srcp, dstp)
    xa, cb, gf = _prep_call(xp, F_w, degs)

    z = xa
    for _ in range(MAX_ITER):
        pf = _spmm_kernel(z.reshape(M * N_PAD), srcp, dstp)
        z = _update_call(pf.reshape(M, N_PAD), cb, xa, gf)

    return _final_call(z, B_w)


# speedup vs baseline: 1.0253x; 1.0253x over previous
import functools

import jax
import jax.numpy as jnp
from jax import lax
from jax.experimental import pallas as pl
from jax.experimental.pallas import tpu as pltpu
from jax.experimental.pallas import tpu_sc as plsc

N = 10000
M = 128
MY = 16
E = 320000
MAX_ITER = 30
GAMMA = 0.8
EPS_F = 1e-6

N_PAD = 10240
CH = 128
N_TILES = 32
CPT = 80
E_PAD = N_TILES * CPT * CH
E_ALLOC = E_PAD + 2 * CH
RPT = N_PAD // 16

_MESH = plsc.VectorSubcoreMesh(core_axis_name="c", subcore_axis_name="s")


@functools.partial(
    pl.kernel,
    out_type=jax.ShapeDtypeStruct((4 * N_PAD, M), jnp.float32),
    mesh=_MESH,
    scratch_types=[
        pltpu.VMEM_SHARED((N_PAD, M), jnp.float32),
        pltpu.VMEM((CH,), jnp.int32),
        pltpu.VMEM((CH, M), jnp.float32),
        pltpu.VMEM((CH, M), jnp.float32),
    ],
)
def _deg_kernel(srcp, dstp, deg_out, acc, idx_v, ones_v, bnc_v):
    cid = lax.axis_index("c")
    sid = lax.axis_index("s")
    wid = sid * 2 + cid
    onerow = jnp.ones((16,), jnp.float32)
    zrow = jnp.zeros((16,), jnp.float32)

    def fill_ones(r, carry):
        for k in range(M // 16):
            ones_v[r, pl.ds(k * 16, 16)] = onerow
        return carry

    lax.fori_loop(0, CH, fill_ones, 0)

    def fill_bnc_zero(r, carry):
        for k in range(M // 16):
            bnc_v[r, pl.ds(k * 16, 16)] = zrow
        return carry

    base_r = sid * RPT

    def zero_acc(b, carry):
        pltpu.sync_copy(bnc_v, acc.at[pl.ds(base_r + b * CH, CH)])
        return carry

    def count_phase(idx_hbm, out_base):
        lax.fori_loop(0, CH, fill_bnc_zero, 0)
        lax.fori_loop(0, RPT // CH, zero_acc, 0)
        plsc.subcore_barrier()

        def edge_step(j, carry):
            base = (wid * CPT + j) * CH
            pltpu.sync_copy(idx_hbm.at[pl.ds(base, CH)], idx_v)
            pltpu.sync_copy(ones_v, acc.at[idx_v], add=True)
            return carry

        lax.fori_loop(0, CPT, edge_step, 0)
        plsc.subcore_barrier()

        def writeback(b, carry):
            off = base_r + b * CH
            pltpu.sync_copy(acc.at[pl.ds(off, CH)], bnc_v)
            pltpu.sync_copy(bnc_v, deg_out.at[pl.ds(out_base + off, CH)])
            return carry

        lax.fori_loop(0, RPT // CH, writeback, 0)
        plsc.subcore_barrier()

    count_phase(srcp, cid * N_PAD)
    count_phase(dstp, 2 * N_PAD + cid * N_PAD)


FPT = M // N_TILES
SG_E = 8192
NSG = E_PAD // SG_E


@functools.partial(
    pl.kernel,
    out_type=jax.ShapeDtypeStruct((M * N_PAD,), jnp.float32),
    mesh=_MESH,
    scratch_types=[
        pltpu.VMEM((FPT * N_PAD,), jnp.float32),
        pltpu.VMEM((FPT * N_PAD,), jnp.float32),
        pltpu.VMEM((SG_E,), jnp.int32),
        pltpu.VMEM((SG_E,), jnp.int32),
        pltpu.VMEM((SG_E,), jnp.int32),
        pltpu.VMEM((SG_E,), jnp.int32),
        pltpu.SemaphoreType.DMA,
        pltpu.SemaphoreType.DMA,
    ],
    compiler_params=pltpu.CompilerParams(needs_layout_passes=False),
)
def _spmm_kernel(zflat, srcp, dstp, pflat, zloc, ploc, ib0s, ib0d,
                 ib1s, ib1d, si0, si1):
    cid = lax.axis_index("c")
    sid = lax.axis_index("s")
    wid = sid * 2 + cid
    IBS, IBD, SI = [ib0s, ib1s], [ib0d, ib1d], [si0, si1]
    zrow = jnp.zeros((16,), jnp.float32)
    my_off = wid * FPT * N_PAD

    pltpu.sync_copy(zflat.at[pl.ds(my_off, FPT * N_PAD)], zloc)

    def fill_zero(r, carry):
        ploc[pl.ds(r * 16, 16)] = zrow
        return carry

    lax.fori_loop(0, FPT * N_PAD // 16, fill_zero, 0)

    def issue(s, sg):
        base = sg * SG_E
        return (pltpu.async_copy(srcp.at[pl.ds(base, SG_E)], IBS[s], SI[s]),
                pltpu.async_copy(dstp.at[pl.ds(base, SG_E)], IBD[s], SI[s]))

    offs = [jnp.full((16,), j * N_PAD, jnp.int32) for j in range(FPT)]
    d = [issue(0, 0), issue(1, 1)]
    for sg in range(NSG):
        s = sg % 2
        d[s][0].wait()
        d[s][1].wait()
        ibs, ibd = IBS[s], IBD[s]

        def edge_group(g, carry):
            off = pl.multiple_of(g * 64, 16)
            svs = [ibs[pl.ds(off + 16 * u, 16)] for u in range(4)]
            dvs = [ibd[pl.ds(off + 16 * u, 16)] for u in range(4)]
            gidx = [svs[u] + offs[j] for u in range(4) for j in range(FPT)]
            vals = [plsc.load_gather(zloc, [gi]) for gi in gidx]
            for u in range(4):
                for j in range(FPT):
                    plsc.addupdate_scatter(ploc, [dvs[u] + offs[j]],
                                           vals[u * FPT + j])
            return carry

        lax.fori_loop(0, SG_E // 64, edge_group, 0)
        if sg + 2 < NSG:
            d[s] = issue(s, sg + 2)

    pltpu.sync_copy(ploc, pflat.at[pl.ds(my_off, FPT * N_PAD)])


def _prep_body(xb, fw, do0, do1, di0, di1, xa_o, cb_o, gf_o):
    dego_t = (do0[...] + do1[...]).T
    degi_t = (di0[...] + di1[...]).T
    a = lax.rsqrt(jnp.maximum(dego_t[0:1, :], 1.0))
    b = lax.rsqrt(jnp.maximum(degi_t[0:1, :], 1.0))
    c = GAMMA * a * b
    xa_o[...] = xb[...] * a
    cb_o[...] = jnp.broadcast_to(c, (M, M))
    g = lax.dot_general(fw[...], fw[...], (((0,), (0,)), ((), ())),
                        precision=lax.Precision.HIGHEST)
    nrm = jnp.sqrt(jnp.sum(g * g))
    gf_o[...] = g / (nrm + EPS_F)


def _prep_call(xp, f_w, degs):
    nb = N_PAD // M
    return pl.pallas_call(
        _prep_body,
        grid=(nb,),
        in_specs=[
            pl.BlockSpec((M, M), lambda i: (0, i)),
            pl.BlockSpec((M, M), lambda i: (0, 0)),
            pl.BlockSpec((M, M), lambda i: (i, 0)),
            pl.BlockSpec((M, M), lambda i, _nb=nb: (i + _nb, 0)),
            pl.BlockSpec((M, M), lambda i, _nb=nb: (i + 2 * _nb, 0)),
            pl.BlockSpec((M, M), lambda i, _nb=nb: (i + 3 * _nb, 0)),
        ],
        out_specs=[
            pl.BlockSpec((M, M), lambda i: (0, i)),
            pl.BlockSpec((M, M), lambda i: (0, i)),
            pl.BlockSpec((M, M), lambda i: (0, 0)),
        ],
        out_shape=[
            jax.ShapeDtypeStruct((M, N_PAD), jnp.float32),
            jax.ShapeDtypeStruct((M, N_PAD), jnp.float32),
            jax.ShapeDtypeStruct((M, M), jnp.float32),
        ],
        compiler_params=pltpu.CompilerParams(
            dimension_semantics=("arbitrary",)),
    )(xp, f_w, degs, degs, degs, degs)


def _update_body(p, cbk, xak, gf, z_o):
    acc = p[...] * cbk[...]
    z_o[...] = lax.dot_general(
        gf[...], acc, (((1,), (0,)), ((), ())),
        precision=lax.Precision.HIGHEST) + xak[...]


def _update_call(pf, cb, xa, gf):
    rb = 2048
    nb = N_PAD // rb
    return pl.pallas_call(
        _update_body,
        grid=(nb,),
        in_specs=[
            pl.BlockSpec((M, rb), lambda i: (0, i)),
            pl.BlockSpec((M, rb), lambda i: (0, i)),
            pl.BlockSpec((M, rb), lambda i: (0, i)),
            pl.BlockSpec((M, M), lambda i: (0, 0)),
        ],
        out_specs=pl.BlockSpec((M, rb), lambda i: (0, i)),
        out_shape=jax.ShapeDtypeStruct((M, N_PAD), jnp.float32),
        compiler_params=pltpu.CompilerParams(
            dimension_semantics=("arbitrary",)),
    )(pf, cb, xa, gf)


def _final_body(zb, bw, o_ref):
    z = zb[...]
    nrm = jnp.maximum(jnp.sqrt(jnp.sum(z * z, axis=0, keepdims=True)), 1e-12)
    zn = z / nrm
    o_ref[...] = lax.dot_general(zn, bw[...], (((0,), (1,)), ((), ())),
                                 precision=lax.Precision.HIGHEST)


def _final_call(z, b_w):
    rb = 1024
    nb = N_PAD // rb
    return pl.pallas_call(
        _final_body,
        grid=(nb,),
        in_specs=[
            pl.BlockSpec((M, rb), lambda i: (0, i)),
            pl.BlockSpec((MY, M), lambda i: (0, 0)),
        ],
        out_specs=pl.BlockSpec((rb, MY), lambda i: (i, 0)),
        out_shape=jax.ShapeDtypeStruct((N, MY), jnp.float32),
        compiler_params=pltpu.CompilerParams(
            dimension_semantics=("arbitrary",)),
    )(z, b_w)


def kernel(X, edge_index, F_w, B_w):
    src = edge_index[0]
    dst = edge_index[1]
    pad = jnp.full((E_PAD - E,), N, dtype=jnp.int32)
    srcp = jnp.concatenate([src, pad])
    dstp = jnp.concatenate([dst, pad])
    xp = jnp.pad(X, ((0, 0), (0, N_PAD - N)))

    degs = _deg_kernel(srcp, dstp)
    xa, cb, gf = _prep_call(xp, F_w, degs)

    z = xa
    for _ in range(MAX_ITER):
        pf = _spmm_kernel(z.reshape(M * N_PAD), srcp, dstp)
        z = _update_call(pf.reshape(M, N_PAD), cb, xa, gf)

    return _final_call(z, B_w)
